# baseline (speedup 1.0000x reference)
import functools

import jax
import jax.numpy as jnp
from jax import lax
from jax.experimental import pallas as pl
from jax.experimental.pallas import tpu as pltpu
from jax.experimental.pallas import tpu_sc as plsc

_N_CELLS = 200
_N_REGIONS = 500
_NSEG = _N_CELLS * _N_REGIONS
_F = 6400000
_NW = 32
_SEG_W = 3136
_NSEG_PAD = _NW * _SEG_W
_CF = 4000
_NCHUNK = _F // _CF
_VPC = _CF // 16
_NSAMP = _NCHUNK
_SVREG = _NSAMP // 16
_GB = 128
_NGB = (_NSAMP + _GB - 1) // _GB

_mesh = plsc.VectorSubcoreMesh(core_axis_name="c", subcore_axis_name="s")


@functools.partial(
    pl.kernel,
    mesh=_mesh,
    out_type=jax.ShapeDtypeStruct((_NSEG_PAD,), jnp.float32),
    scratch_types=[
        pltpu.VMEM((_SEG_W,), jnp.float32),
        pltpu.VMEM((_CF,), jnp.float32),
        pltpu.VMEM((_CF,), jnp.float32),
        pltpu.VMEM((_CF + 16,), jnp.int32),
        pltpu.VMEM((_CF + 16,), jnp.int32),
        pltpu.VMEM((_NGB * _GB,), jnp.int32),
        pltpu.VMEM((_NGB * _GB,), jnp.int32),
        pltpu.SemaphoreType.DMA,
        pltpu.SemaphoreType.DMA,
        pltpu.SemaphoreType.DMA,
        pltpu.SemaphoreType.DMA,
        pltpu.SemaphoreType.DMA,
    ],
    compiler_params=pltpu.CompilerParams(needs_layout_passes=False),
)
def _sc_segment_sum(vals_hbm, idx_hbm, cnt_hbm, out_hbm, acc_v,
                    vals0, vals1, idx0, idx1, spos, samp,
                    sv0, sv1, si0, si1, sg):
    c = lax.axis_index("c")
    s = lax.axis_index("s")
    wid = s * 2 + c
    b_lo = wid * _SEG_W
    b_hi = b_lo + _SEG_W
    svs = (sv0, sv1)
    sis = (si0, si1)
    vbufs = (vals0, vals1)
    ibufs = (idx0, idx1)

    lane = lax.iota(jnp.int32, 16)
    is_last_lane = lane == 15

    pltpu.sync_copy(cnt_hbm.at[pl.ds(b_lo, _SEG_W)], acc_v)

    def _fill_pos(i, carry):
        k = i * 16 + lane
        pos = jnp.where(k < _NSAMP, k * _CF, 0)
        spos[pl.ds(i * 16, 16)] = pos
        return carry

    lax.fori_loop(0, _NGB * _GB // 16, _fill_pos, 0, unroll=8)

    def _gather_copies():
        return [
            pltpu.make_async_copy(
                idx_hbm.at[spos.at[pl.ds(j * _GB, _GB)]],
                samp.at[pl.ds(j * _GB, _GB)],
                sg,
            )
            for j in range(_NGB)
        ]

    for cp in _gather_copies():
        cp.start()
    for cp in _gather_copies():
        cp.wait()

    def _count(i, carry):
        cl, ch = carry
        sv = samp[pl.ds(i * 16, 16)]
        one = jnp.ones((16,), jnp.int32)
        zero = jnp.zeros((16,), jnp.int32)
        cl = cl + jnp.where(sv < b_lo, one, zero)
        ch = ch + jnp.where(sv < b_hi, one, zero)
        return cl, ch

    zeros_i = jnp.zeros((16,), jnp.int32)
    cl, ch = lax.fori_loop(0, _SVREG, _count, (zeros_i, zeros_i), unroll=8)
    k_lo = jnp.sum(cl)
    k_hi = jnp.sum(ch)
    j_lo = jnp.maximum(k_lo - 1, 0)
    j_hi = k_hi

    def _copies(g, b):
        off = g * _CF
        cv = pltpu.make_async_copy(
            vals_hbm.at[pl.ds(off, _CF)], vbufs[b], svs[b])
        ci = pltpu.make_async_copy(
            idx_hbm.at[pl.ds(off, _CF)], ibufs[b].at[pl.ds(0, _CF)], sis[b])
        return cv, ci

    def _start(g, b):
        cv, ci = _copies(g, b)
        cv.start()
        ci.start()

    def _wait(g, b):
        cv, ci = _copies(g, b)
        cv.wait()
        ci.wait()

    def _compute(g, b):
        _wait(g, b)

        @plsc.parallel_loop(0, _VPC, unroll=8)
        def _vreg(i):
            ix = ibufs[b][pl.ds(i * 16, 16)]
            nx = ibufs[b][pl.ds(i * 16 + 1, 16)]
            x = vbufs[b][pl.ds(i * 16, 16)]
            csum = plsc.cumsum(x)
            boundary = ix != nx
            inr = (ix >= b_lo) & (ix < b_hi)
            nxr = (nx >= b_lo) & (nx < b_hi)
            m_end = (boundary | is_last_lane) & inr
            m_carry = boundary & (~is_last_lane) & nxr
            plsc.addupdate_scatter(acc_v, [ix - b_lo], csum, mask=m_end)
            plsc.addupdate_scatter(acc_v, [nx - b_lo], -csum, mask=m_carry)

    izeros = jnp.zeros((16,), jnp.int32)
    idx0[pl.ds(_CF, 16)] = izeros
    idx1[pl.ds(_CF, 16)] = izeros

    @pl.when(j_lo < j_hi)
    def _():
        _start(j_lo, 0)

    def _outer(gg, carry):
        g0 = j_lo + gg * 2

        @pl.when(g0 + 1 < j_hi)
        def _():
            _start(g0 + 1, 1)

        _compute(g0, 0)

        @pl.when(g0 + 2 < j_hi)
        def _():
            _start(g0 + 2, 0)

        @pl.when(g0 + 1 < j_hi)
        def _():
            _compute(g0 + 1, 1)

        return carry

    lax.fori_loop(0, (j_hi - j_lo + 1) // 2, _outer, 0)

    pltpu.sync_copy(acc_v, out_hbm.at[pl.ds(b_lo, _SEG_W)])


def kernel(likelihood_position, likelihood_count, local_cellxregion_ix):
    cnt = jnp.pad(likelihood_count, (0, _NSEG_PAD - _NSEG))
    out = _sc_segment_sum(likelihood_position, local_cellxregion_ix, cnt)
    return out[:_NSEG].reshape(_N_CELLS, _N_REGIONS)


# speedup vs baseline: 1.2333x; 1.2333x over previous
import functools

import jax
import jax.numpy as jnp
from jax import lax
from jax.experimental import pallas as pl
from jax.experimental.pallas import tpu as pltpu
from jax.experimental.pallas import tpu_sc as plsc

_N_CELLS = 200
_N_REGIONS = 500
_NSEG = _N_CELLS * _N_REGIONS
_F = 6400000
_NW = 32
_SEG_W = 3136
_NSEG_PAD = _NW * _SEG_W
_CF = 4000
_NCHUNK = _F // _CF
_VPC = _CF // 16
_NSAMP = _NCHUNK
_SVREG = _NSAMP // 16
_GB = 128
_NGB = (_NSAMP + _GB - 1) // _GB

_mesh = plsc.VectorSubcoreMesh(core_axis_name="c", subcore_axis_name="s")


@functools.partial(
    pl.kernel,
    mesh=_mesh,
    out_type=jax.ShapeDtypeStruct((_NSEG_PAD,), jnp.float32),
    scratch_types=[
        pltpu.VMEM((4 * _SEG_W,), jnp.float32),
        pltpu.VMEM((_CF,), jnp.float32),
        pltpu.VMEM((_CF,), jnp.float32),
        pltpu.VMEM((_CF + 16,), jnp.int32),
        pltpu.VMEM((_CF + 16,), jnp.int32),
        pltpu.VMEM((_NGB * _GB,), jnp.int32),
        pltpu.VMEM((_NGB * _GB,), jnp.int32),
        pltpu.SemaphoreType.DMA,
        pltpu.SemaphoreType.DMA,
        pltpu.SemaphoreType.DMA,
        pltpu.SemaphoreType.DMA,
        pltpu.SemaphoreType.DMA,
    ],
    compiler_params=pltpu.CompilerParams(needs_layout_passes=False),
)
def _sc_segment_sum(vals_hbm, idx_hbm, cnt_hbm, out_hbm, acc_v,
                    vals0, vals1, idx0, idx1, spos, samp,
                    sv0, sv1, si0, si1, sg):
    c = lax.axis_index("c")
    s = lax.axis_index("s")
    wid = s * 2 + c
    b_lo = wid * _SEG_W
    b_hi = b_lo + _SEG_W
    svs = (sv0, sv1)
    sis = (si0, si1)
    vbufs = (vals0, vals1)
    ibufs = (idx0, idx1)

    lane = lax.iota(jnp.int32, 16)
    is_last_lane = lane == 15

    pltpu.sync_copy(cnt_hbm.at[pl.ds(b_lo, _SEG_W)], acc_v.at[pl.ds(0, _SEG_W)])
    zeros_f = jnp.zeros((16,), jnp.float32)

    def _zero(i, carry):
        acc_v[pl.ds(_SEG_W + i * 16, 16)] = zeros_f
        return carry

    lax.fori_loop(0, 3 * _SEG_W // 16, _zero, 0, unroll=8)

    def _fill_pos(i, carry):
        k = i * 16 + lane
        pos = jnp.where(k < _NSAMP, k * _CF, 0)
        spos[pl.ds(i * 16, 16)] = pos
        return carry

    lax.fori_loop(0, _NGB * _GB // 16, _fill_pos, 0, unroll=8)

    def _gather_copies():
        return [
            pltpu.make_async_copy(
                idx_hbm.at[spos.at[pl.ds(j * _GB, _GB)]],
                samp.at[pl.ds(j * _GB, _GB)],
                sg,
            )
            for j in range(_NGB)
        ]

    for cp in _gather_copies():
        cp.start()
    for cp in _gather_copies():
        cp.wait()

    def _count(i, carry):
        cl, ch = carry
        sv = samp[pl.ds(i * 16, 16)]
        one = jnp.ones((16,), jnp.int32)
        zero = jnp.zeros((16,), jnp.int32)
        cl = cl + jnp.where(sv < b_lo, one, zero)
        ch = ch + jnp.where(sv < b_hi, one, zero)
        return cl, ch

    zeros_i = jnp.zeros((16,), jnp.int32)
    cl, ch = lax.fori_loop(0, _SVREG, _count, (zeros_i, zeros_i), unroll=8)
    k_lo = jnp.sum(cl)
    k_hi = jnp.sum(ch)
    j_lo = jnp.maximum(k_lo - 1, 0)
    j_hi = k_hi

    def _copies(g, b):
        off = g * _CF
        cv = pltpu.make_async_copy(
            vals_hbm.at[pl.ds(off, _CF)], vbufs[b], svs[b])
        ci = pltpu.make_async_copy(
            idx_hbm.at[pl.ds(off, _CF)], ibufs[b].at[pl.ds(0, _CF)], sis[b])
        return cv, ci

    def _start(g, b):
        cv, ci = _copies(g, b)
        cv.start()
        ci.start()

    def _wait(g, b):
        cv, ci = _copies(g, b)
        cv.wait()
        ci.wait()

    def _compute(g, b):
        _wait(g, b)

        @plsc.parallel_loop(0, _VPC, unroll=8)
        def _vreg(i):
            bank = (i & 3) * _SEG_W
            ix = ibufs[b][pl.ds(i * 16, 16)]
            nx = ibufs[b][pl.ds(i * 16 + 1, 16)]
            x = vbufs[b][pl.ds(i * 16, 16)]
            csum = plsc.cumsum(x)
            boundary = ix != nx
            lix = ix - b_lo
            lnx = nx - b_lo
            inr = plsc.bitcast(lix, jnp.uint32) < jnp.uint32(_SEG_W)
            nxr = plsc.bitcast(lnx, jnp.uint32) < jnp.uint32(_SEG_W)
            m_end = (boundary | is_last_lane) & inr
            m_carry = boundary & (~is_last_lane) & nxr
            plsc.addupdate_scatter(acc_v, [lix + bank], csum, mask=m_end)
            plsc.addupdate_scatter(acc_v, [lnx + bank], -csum, mask=m_carry)

    izeros = jnp.zeros((16,), jnp.int32)
    idx0[pl.ds(_CF, 16)] = izeros
    idx1[pl.ds(_CF, 16)] = izeros

    @pl.when(j_lo < j_hi)
    def _():
        _start(j_lo, 0)

    def _outer(gg, carry):
        g0 = j_lo + gg * 2

        @pl.when(g0 + 1 < j_hi)
        def _():
            _start(g0 + 1, 1)

        _compute(g0, 0)

        @pl.when(g0 + 2 < j_hi)
        def _():
            _start(g0 + 2, 0)

        @pl.when(g0 + 1 < j_hi)
        def _():
            _compute(g0 + 1, 1)

        return carry

    lax.fori_loop(0, (j_hi - j_lo + 1) // 2, _outer, 0)

    @plsc.parallel_loop(0, _SEG_W // 16, unroll=8)
    def _fold(i):
        o = i * 16
        acc_v[pl.ds(o, 16)] = (
            acc_v[pl.ds(o, 16)]
            + acc_v[pl.ds(_SEG_W + o, 16)]
            + acc_v[pl.ds(2 * _SEG_W + o, 16)]
            + acc_v[pl.ds(3 * _SEG_W + o, 16)]
        )

    pltpu.sync_copy(acc_v.at[pl.ds(0, _SEG_W)], out_hbm.at[pl.ds(b_lo, _SEG_W)])


def kernel(likelihood_position, likelihood_count, local_cellxregion_ix):
    cnt = jnp.pad(likelihood_count, (0, _NSEG_PAD - _NSEG))
    out = _sc_segment_sum(likelihood_position, local_cellxregion_ix, cnt)
    return out[:_NSEG].reshape(_N_CELLS, _N_REGIONS)


# speedup vs baseline: 1.2855x; 1.0423x over previous
import functools

import jax
import jax.numpy as jnp
from jax import lax
from jax.experimental import pallas as pl
from jax.experimental.pallas import tpu as pltpu
from jax.experimental.pallas import tpu_sc as plsc

_N_CELLS = 200
_N_REGIONS = 500
_NSEG = _N_CELLS * _N_REGIONS
_F = 6400000
_NW = 32
_SEG_W = 3136
_NSEG_PAD = _NW * _SEG_W
_CF = 4000
_NCHUNK = _F // _CF
_VPC = _CF // 16
_NSAMP = _NCHUNK
_SVREG = _NSAMP // 16
_GB = 128
_NGB = (_NSAMP + _GB - 1) // _GB

_mesh = plsc.VectorSubcoreMesh(core_axis_name="c", subcore_axis_name="s")


@functools.partial(
    pl.kernel,
    mesh=_mesh,
    out_type=jax.ShapeDtypeStruct((_NSEG_PAD,), jnp.float32),
    scratch_types=[
        pltpu.VMEM((4 * _SEG_W,), jnp.float32),
        pltpu.VMEM((_CF,), jnp.float32),
        pltpu.VMEM((_CF,), jnp.float32),
        pltpu.VMEM((_CF + 16,), jnp.int32),
        pltpu.VMEM((_CF + 16,), jnp.int32),
        pltpu.VMEM((_NGB * _GB,), jnp.int32),
        pltpu.VMEM((_NGB * _GB,), jnp.int32),
        pltpu.SemaphoreType.DMA,
        pltpu.SemaphoreType.DMA,
        pltpu.SemaphoreType.DMA,
        pltpu.SemaphoreType.DMA,
        pltpu.SemaphoreType.DMA,
    ],
    compiler_params=pltpu.CompilerParams(needs_layout_passes=False),
)
def _sc_segment_sum(vals_hbm, idx_hbm, cnt_hbm, out_hbm, acc_v,
                    vals0, vals1, idx0, idx1, spos, samp,
                    sv0, sv1, si0, si1, sg):
    c = lax.axis_index("c")
    s = lax.axis_index("s")
    wid = s * 2 + c
    b_lo = wid * _SEG_W
    b_hi = b_lo + _SEG_W
    svs = (sv0, sv1)
    sis = (si0, si1)
    vbufs = (vals0, vals1)
    ibufs = (idx0, idx1)

    lane = lax.iota(jnp.int32, 16)
    is_last_lane = lane == 15

    pltpu.sync_copy(cnt_hbm.at[pl.ds(b_lo, _SEG_W)], acc_v.at[pl.ds(0, _SEG_W)])
    zeros_f = jnp.zeros((16,), jnp.float32)

    def _zero(i, carry):
        acc_v[pl.ds(_SEG_W + i * 16, 16)] = zeros_f
        return carry

    lax.fori_loop(0, 3 * _SEG_W // 16, _zero, 0, unroll=8)

    def _fill_pos(i, carry):
        k = i * 16 + lane
        pos = jnp.where(k < _NSAMP, k * _CF, 0)
        spos[pl.ds(i * 16, 16)] = pos
        return carry

    lax.fori_loop(0, _NGB * _GB // 16, _fill_pos, 0, unroll=8)

    def _gather_copies():
        return [
            pltpu.make_async_copy(
                idx_hbm.at[spos.at[pl.ds(j * _GB, _GB)]],
                samp.at[pl.ds(j * _GB, _GB)],
                sg,
            )
            for j in range(_NGB)
        ]

    for cp in _gather_copies():
        cp.start()
    for cp in _gather_copies():
        cp.wait()

    def _count(i, carry):
        cl, ch = carry
        sv = samp[pl.ds(i * 16, 16)]
        one = jnp.ones((16,), jnp.int32)
        zero = jnp.zeros((16,), jnp.int32)
        cl = cl + jnp.where(sv < b_lo, one, zero)
        ch = ch + jnp.where(sv < b_hi, one, zero)
        return cl, ch

    zeros_i = jnp.zeros((16,), jnp.int32)
    cl, ch = lax.fori_loop(0, _SVREG, _count, (zeros_i, zeros_i), unroll=8)
    k_lo = jnp.sum(cl)
    k_hi = jnp.sum(ch)
    j_lo = jnp.maximum(k_lo - 1, 0)
    j_hi = k_hi

    def _copies(g, b):
        off = g * _CF
        cv = pltpu.make_async_copy(
            vals_hbm.at[pl.ds(off, _CF)], vbufs[b], svs[b])
        ci = pltpu.make_async_copy(
            idx_hbm.at[pl.ds(off, _CF)], ibufs[b].at[pl.ds(0, _CF)], sis[b])
        return cv, ci

    def _start(g, b):
        cv, ci = _copies(g, b)
        cv.start()
        ci.start()

    def _wait(g, b):
        cv, ci = _copies(g, b)
        cv.wait()
        ci.wait()

    def _compute(g, b):
        _wait(g, b)

        @plsc.parallel_loop(0, _VPC, unroll=4)
        def _vreg(i):
            bank = (i & 3) * _SEG_W
            ix = ibufs[b][pl.ds(i * 16, 16)]
            nx = ibufs[b][pl.ds(i * 16 + 1, 16)]
            x = vbufs[b][pl.ds(i * 16, 16)]
            csum = plsc.cumsum(x)
            boundary = ix != nx
            lix = ix - b_lo
            lnx = nx - b_lo
            inr = plsc.bitcast(lix, jnp.uint32) < jnp.uint32(_SEG_W)
            nxr = plsc.bitcast(lnx, jnp.uint32) < jnp.uint32(_SEG_W)
            m_end = (boundary | is_last_lane) & inr
            m_carry = boundary & (~is_last_lane) & nxr
            plsc.addupdate_scatter(acc_v, [lix + bank], csum, mask=m_end)
            plsc.addupdate_scatter(acc_v, [lnx + bank], -csum, mask=m_carry)

    izeros = jnp.zeros((16,), jnp.int32)
    idx0[pl.ds(_CF, 16)] = izeros
    idx1[pl.ds(_CF, 16)] = izeros

    @pl.when(j_lo < j_hi)
    def _():
        _start(j_lo, 0)

    def _outer(gg, carry):
        g0 = j_lo + gg * 2

        @pl.when(g0 + 1 < j_hi)
        def _():
            _start(g0 + 1, 1)

        _compute(g0, 0)

        @pl.when(g0 + 2 < j_hi)
        def _():
            _start(g0 + 2, 0)

        @pl.when(g0 + 1 < j_hi)
        def _():
            _compute(g0 + 1, 1)

        return carry

    lax.fori_loop(0, (j_hi - j_lo + 1) // 2, _outer, 0)

    @plsc.parallel_loop(0, _SEG_W // 16, unroll=8)
    def _fold(i):
        o = i * 16
        acc_v[pl.ds(o, 16)] = (
            acc_v[pl.ds(o, 16)]
            + acc_v[pl.ds(_SEG_W + o, 16)]
            + acc_v[pl.ds(2 * _SEG_W + o, 16)]
            + acc_v[pl.ds(3 * _SEG_W + o, 16)]
        )

    pltpu.sync_copy(acc_v.at[pl.ds(0, _SEG_W)], out_hbm.at[pl.ds(b_lo, _SEG_W)])


def kernel(likelihood_position, likelihood_count, local_cellxregion_ix):
    cnt = jnp.pad(likelihood_count, (0, _NSEG_PAD - _NSEG))
    out = _sc_segment_sum(likelihood_position, local_cellxregion_ix, cnt)
    return out[:_NSEG].reshape(_N_CELLS, _N_REGIONS)


# speedup vs baseline: 1.3035x; 1.0140x over previous
import functools

import jax
import jax.numpy as jnp
from jax import lax
from jax.experimental import pallas as pl
from jax.experimental.pallas import tpu as pltpu
from jax.experimental.pallas import tpu_sc as plsc

_N_CELLS = 200
_N_REGIONS = 500
_NSEG = _N_CELLS * _N_REGIONS
_F = 6400000
_NW = 32
_SEG_W = 3136
_NSEG_PAD = _NW * _SEG_W
_CF = 4000
_NCHUNK = _F // _CF
_VPC = _CF // 16
_NSAMP = _NCHUNK
_SVREG = _NSAMP // 16
_GB = 128
_NGB = (_NSAMP + _GB - 1) // _GB

_mesh = plsc.VectorSubcoreMesh(core_axis_name="c", subcore_axis_name="s")


@functools.partial(
    pl.kernel,
    mesh=_mesh,
    out_type=jax.ShapeDtypeStruct((_NSEG_PAD,), jnp.float32),
    scratch_types=[
        pltpu.VMEM((_SEG_W,), jnp.float32),
        pltpu.VMEM((_CF,), jnp.float32),
        pltpu.VMEM((_CF,), jnp.float32),
        pltpu.VMEM((_CF + 16,), jnp.int32),
        pltpu.VMEM((_CF + 16,), jnp.int32),
        pltpu.VMEM((_NGB * _GB,), jnp.int32),
        pltpu.VMEM((_NGB * _GB,), jnp.int32),
        pltpu.SemaphoreType.DMA,
        pltpu.SemaphoreType.DMA,
        pltpu.SemaphoreType.DMA,
        pltpu.SemaphoreType.DMA,
        pltpu.SemaphoreType.DMA,
    ],
    compiler_params=pltpu.CompilerParams(needs_layout_passes=False),
)
def _sc_segment_sum(vals_hbm, idx_hbm, cnt_hbm, out_hbm, acc_v,
                    vals0, vals1, idx0, idx1, spos, samp,
                    sv0, sv1, si0, si1, sg):
    c = lax.axis_index("c")
    s = lax.axis_index("s")
    wid = s * 2 + c
    b_lo = wid * _SEG_W
    b_hi = b_lo + _SEG_W
    svs = (sv0, sv1)
    sis = (si0, si1)
    vbufs = (vals0, vals1)
    ibufs = (idx0, idx1)

    lane = lax.iota(jnp.int32, 16)
    is_last_lane = lane == 15

    pltpu.sync_copy(cnt_hbm.at[pl.ds(b_lo, _SEG_W)], acc_v)

    def _fill_pos(i, carry):
        k = i * 16 + lane
        pos = jnp.where(k < _NSAMP, k * _CF, 0)
        spos[pl.ds(i * 16, 16)] = pos
        return carry

    lax.fori_loop(0, _NGB * _GB // 16, _fill_pos, 0, unroll=8)

    def _gather_copies():
        return [
            pltpu.make_async_copy(
                idx_hbm.at[spos.at[pl.ds(j * _GB, _GB)]],
                samp.at[pl.ds(j * _GB, _GB)],
                sg,
            )
            for j in range(_NGB)
        ]

    for cp in _gather_copies():
        cp.start()
    for cp in _gather_copies():
        cp.wait()

    def _count(i, carry):
        cl, ch = carry
        sv = samp[pl.ds(i * 16, 16)]
        one = jnp.ones((16,), jnp.int32)
        zero = jnp.zeros((16,), jnp.int32)
        cl = cl + jnp.where(sv < b_lo, one, zero)
        ch = ch + jnp.where(sv < b_hi, one, zero)
        return cl, ch

    zeros_i = jnp.zeros((16,), jnp.int32)
    cl, ch = lax.fori_loop(0, _SVREG, _count, (zeros_i, zeros_i), unroll=8)
    k_lo = jnp.sum(cl)
    k_hi = jnp.sum(ch)
    j_lo = jnp.maximum(k_lo - 1, 0)
    j_hi = k_hi

    def _copies(g, b):
        off = g * _CF
        cv = pltpu.make_async_copy(
            vals_hbm.at[pl.ds(off, _CF)], vbufs[b], svs[b])
        ci = pltpu.make_async_copy(
            idx_hbm.at[pl.ds(off, _CF)], ibufs[b].at[pl.ds(0, _CF)], sis[b])
        return cv, ci

    def _start(g, b):
        cv, ci = _copies(g, b)
        cv.start()
        ci.start()

    def _wait(g, b):
        cv, ci = _copies(g, b)
        cv.wait()
        ci.wait()

    def _compute(g, b):
        _wait(g, b)

        @plsc.parallel_loop(0, _VPC, unroll=4)
        def _vreg(i):
            ix = ibufs[b][pl.ds(i * 16, 16)]
            nx = ibufs[b][pl.ds(i * 16 + 1, 16)]
            x = vbufs[b][pl.ds(i * 16, 16)]
            csum = plsc.cumsum(x)
            boundary = ix != nx
            lix = ix - b_lo
            lnx = nx - b_lo
            inr = plsc.bitcast(lix, jnp.uint32) < jnp.uint32(_SEG_W)
            nxr = plsc.bitcast(lnx, jnp.uint32) < jnp.uint32(_SEG_W)
            m_end = (boundary | is_last_lane) & inr
            m_carry = boundary & (~is_last_lane) & nxr
            plsc.addupdate_scatter(acc_v, [lix], csum, mask=m_end)
            plsc.addupdate_scatter(acc_v, [lnx], -csum, mask=m_carry)

    izeros = jnp.zeros((16,), jnp.int32)
    idx0[pl.ds(_CF, 16)] = izeros
    idx1[pl.ds(_CF, 16)] = izeros

    @pl.when(j_lo < j_hi)
    def _():
        _start(j_lo, 0)

    def _outer(gg, carry):
        g0 = j_lo + gg * 2

        @pl.when(g0 + 1 < j_hi)
        def _():
            _start(g0 + 1, 1)

        _compute(g0, 0)

        @pl.when(g0 + 2 < j_hi)
        def _():
            _start(g0 + 2, 0)

        @pl.when(g0 + 1 < j_hi)
        def _():
            _compute(g0 + 1, 1)

        return carry

    lax.fori_loop(0, (j_hi - j_lo + 1) // 2, _outer, 0)

    pltpu.sync_copy(acc_v, out_hbm.at[pl.ds(b_lo, _SEG_W)])


def kernel(likelihood_position, likelihood_count, local_cellxregion_ix):
    cnt = jnp.pad(likelihood_count, (0, _NSEG_PAD - _NSEG))
    out = _sc_segment_sum(likelihood_position, local_cellxregion_ix, cnt)
    return out[:_NSEG].reshape(_N_CELLS, _N_REGIONS)
